# baseline (device time: 208957 ns/iter reference)
import jax
import jax.numpy as jnp
from jax import lax
from jax.experimental import pallas as pl
from jax.experimental.pallas import tpu as pltpu

NP = 8
NSUB = 2
RPP = 704
RP = NSUB * RPP
N_CW = 4
N_CCW = 3

RING_ROWS = NP * RP
N_ROT = 4
N_RDRAIN = 8


def _ring_pos(y, z):
    return jnp.where(y == 0, z, 7 - z)


def _ring_coords(p):
    y = jnp.where(p < 4, 0, 1)
    z = jnp.where(p < 4, p, 7 - p)
    return y, z


def kernel(x):
    m, n = x.shape
    rem_rows = m - RING_ROWS
    n_full = m // RPP
    tail = m - n_full * RPP
    chunk_lens = [RPP] * n_full + ([tail] if tail else [])
    n_chunks = len(chunk_lens)

    def body(
        x_ref,
        out_ref,
        ring_buf,
        inj_stage,
        rem_stage,
        rot_buf,
        ld_buf,
        ld_sems,
        inj_send_sems,
        inj_recv_sems,
        rem_send_sem,
        rem_recv_sem,
        cw_send_sems,
        cw_recv_sems,
        ccw_send_sems,
        ccw_recv_sems,
        ring_drain_sems,
        own_drain_sems,
        dummy_sem,
    ):
        my_x = lax.axis_index("x")
        my_y = lax.axis_index("y")
        my_z = lax.axis_index("z")
        other = 1 - my_x
        partner = (other, my_y, my_z)
        my_r = _ring_pos(my_y, my_z)
        cw_y, cw_z = _ring_coords((my_r + 1) % NP)
        ccw_y, ccw_z = _ring_coords((my_r - 1) % NP)
        cw_dev = (my_x, cw_y, cw_z)
        ccw_dev = (my_x, ccw_y, ccw_z)

        def sub_slice(piece, j):
            return pl.ds((piece * NSUB + j) * RPP, RPP)

        inj_lds = []
        for j in range(NSUB):
            ld = pltpu.make_async_copy(
                x_ref.at[pl.ds(my_r * RP + j * RPP, RPP), :],
                ld_buf.at[j],
                ld_sems.at[j],
            )
            ld.start()
            inj_lds.append(ld)

        barrier_sem = pltpu.get_barrier_semaphore()
        for dev in (partner, cw_dev, ccw_dev):
            pl.semaphore_signal(
                barrier_sem, inc=1,
                device_id=dev, device_id_type=pl.DeviceIdType.MESH,
            )
        pl.semaphore_wait(barrier_sem, 3)

        injs = []
        for j in range(NSUB):
            inj_lds[j].wait()
            inj_stage[pl.ds(j * RPP, RPP), :] = ld_buf[j].astype(jnp.bfloat16)
            r = pltpu.make_async_remote_copy(
                src_ref=inj_stage.at[pl.ds(j * RPP, RPP), :],
                dst_ref=ring_buf.at[sub_slice(my_r, j), :],
                send_sem=inj_send_sems.at[j],
                recv_sem=inj_recv_sems.at[j],
                device_id=partner,
                device_id_type=pl.DeviceIdType.MESH,
            )
            r.start()
            injs.append(r)

        order = list(range(2 * NP, n_chunks)) + list(range(2 * NP))
        chunk_lds = {}
        own_drains = []
        rem_rdma = None

        def start_ld(i):
            k = order[i]
            ln = chunk_lens[k]
            ld = pltpu.make_async_copy(
                x_ref.at[pl.ds(k * RPP, ln), :],
                ld_buf.at[i % 2, pl.ds(0, ln), :],
                ld_sems.at[i % 2],
            )
            ld.start()
            chunk_lds[i] = ld

        start_ld(0)
        start_ld(1)
        for i in range(n_chunks):
            k = order[i]
            ln = chunk_lens[k]
            chunk_lds[i].wait()
            if i >= N_ROT:
                own_drains[i - N_ROT].wait()
            if k < 2 * NP:
                rot_buf[i % N_ROT, :, :] = ld_buf[i % 2].astype(jnp.bfloat16)
                src = rot_buf.at[i % N_ROT]
            else:
                off = (k - 2 * NP) * RPP
                rem_stage[pl.ds(off, ln), :] = (
                    ld_buf[i % 2, pl.ds(0, ln), :].astype(jnp.bfloat16)
                )
                src = rem_stage.at[pl.ds(off, ln), :]
            if i + 2 < n_chunks:
                start_ld(i + 2)
            d = pltpu.make_async_copy(
                src,
                out_ref.at[pl.ds(my_x * m + k * RPP, ln), :],
                own_drain_sems.at[i % N_ROT],
            )
            d.start()
            own_drains.append(d)
            if i == n_chunks - 2 * NP - 1:
                rem_rdma = pltpu.make_async_remote_copy(
                    src_ref=rem_stage,
                    dst_ref=out_ref.at[
                        pl.ds(my_x * m + RING_ROWS, rem_rows), :
                    ],
                    send_sem=rem_send_sem,
                    recv_sem=rem_recv_sem,
                    device_id=partner,
                    device_id_type=pl.DeviceIdType.MESH,
                )
                rem_rdma.start()

        def stream_send(sem_s, sem_r, dev, piece, s, j):
            r = pltpu.make_async_remote_copy(
                src_ref=ring_buf.at[sub_slice(piece, j), :],
                dst_ref=ring_buf.at[sub_slice(piece, j), :],
                send_sem=sem_s.at[s * NSUB + j],
                recv_sem=sem_r.at[s * NSUB + j],
                device_id=dev,
                device_id_type=pl.DeviceIdType.MESH,
            )
            r.start()
            return r

        def recv_wait(sem_arr, s, j, piece):
            pltpu.make_async_remote_copy(
                src_ref=ring_buf.at[sub_slice(piece, j), :],
                dst_ref=ring_buf.at[sub_slice(piece, j), :],
                send_sem=dummy_sem,
                recv_sem=sem_arr.at[s * NSUB + j],
                device_id=cw_dev,
                device_id_type=pl.DeviceIdType.MESH,
            ).wait_recv()

        ring_drains = []

        def ring_drain(piece, j):
            i = len(ring_drains)
            if i >= N_RDRAIN:
                ring_drains[i - N_RDRAIN].wait()
            d = pltpu.make_async_copy(
                ring_buf.at[sub_slice(piece, j), :],
                out_ref.at[pl.ds(other * m + (piece * NSUB + j) * RPP, RPP), :],
                ring_drain_sems.at[i % N_RDRAIN],
            )
            d.start()
            ring_drains.append(d)

        sends = []
        for j in range(NSUB):
            injs[j].wait_recv()
            ring_drain(my_r, j)
            sends.append(stream_send(
                cw_send_sems, cw_recv_sems, cw_dev, my_r, 0, j))
            sends.append(stream_send(
                ccw_send_sems, ccw_recv_sems, ccw_dev, my_r, 0, j))
        for s in range(1, N_CW):
            for j in range(NSUB):
                q = (my_r - s) % NP
                recv_wait(cw_recv_sems, s - 1, j, q)
                sends.append(stream_send(
                    cw_send_sems, cw_recv_sems, cw_dev, q, s, j))
                ring_drain(q, j)
                if s < N_CCW:
                    qq = (my_r + s) % NP
                    recv_wait(ccw_recv_sems, s - 1, j, qq)
                    sends.append(stream_send(
                        ccw_send_sems, ccw_recv_sems, ccw_dev, qq, s, j))
                    ring_drain(qq, j)
        for j in range(NSUB):
            recv_wait(cw_recv_sems, N_CW - 1, j, (my_r - N_CW) % NP)
            ring_drain((my_r - N_CW) % NP, j)
            recv_wait(ccw_recv_sems, N_CCW - 1, j, (my_r + N_CCW) % NP)
            ring_drain((my_r + N_CCW) % NP, j)

        rem_rdma.wait_recv()

        for r in injs:
            r.wait_send()
        rem_rdma.wait_send()
        for snd in sends:
            snd.wait_send()
        for d in own_drains[-N_ROT:]:
            d.wait()
        for d in ring_drains[-N_RDRAIN:]:
            d.wait()

    out_shape = jax.ShapeDtypeStruct((2 * m, n), jnp.bfloat16)
    return pl.pallas_call(
        body,
        out_shape=out_shape,
        in_specs=[pl.BlockSpec(memory_space=pl.ANY)],
        out_specs=pl.BlockSpec(memory_space=pl.ANY),
        scratch_shapes=[
            pltpu.VMEM((RING_ROWS, n), jnp.bfloat16),
            pltpu.VMEM((RP, n), jnp.bfloat16),
            pltpu.VMEM((m - RING_ROWS, n), jnp.bfloat16),
            pltpu.VMEM((N_ROT, RPP, n), jnp.bfloat16),
            pltpu.VMEM((2, RPP, n), jnp.float32),
            pltpu.SemaphoreType.DMA((2,)),
            pltpu.SemaphoreType.DMA((NSUB,)),
            pltpu.SemaphoreType.DMA((NSUB,)),
            pltpu.SemaphoreType.DMA,
            pltpu.SemaphoreType.DMA,
            pltpu.SemaphoreType.DMA((N_CW * NSUB,)),
            pltpu.SemaphoreType.DMA((N_CW * NSUB,)),
            pltpu.SemaphoreType.DMA((N_CCW * NSUB,)),
            pltpu.SemaphoreType.DMA((N_CCW * NSUB,)),
            pltpu.SemaphoreType.DMA((N_RDRAIN,)),
            pltpu.SemaphoreType.DMA((N_ROT,)),
            pltpu.SemaphoreType.DMA,
        ],
        compiler_params=pltpu.CompilerParams(
            collective_id=0, vmem_limit_bytes=56 * 1024 * 1024
        ),
    )(x)


# device time: 195080 ns/iter; 1.0711x vs baseline; 1.0711x over previous
import jax
import jax.numpy as jnp
from jax import lax
from jax.experimental import pallas as pl
from jax.experimental.pallas import tpu as pltpu

NP = 8
NSUB = 2
RPP = 704
RP = NSUB * RPP
N_CW = 4
N_CCW = 3

RING_ROWS = NP * RP
N_ROT = 4
N_RDRAIN = 8


def _ring_pos(y, z):
    return jnp.where(y == 0, z, 7 - z)


def _ring_coords(p):
    y = jnp.where(p < 4, 0, 1)
    z = jnp.where(p < 4, p, 7 - p)
    return y, z


def kernel(x):
    m, n = x.shape
    rem_rows = m - RING_ROWS
    n_full = m // RPP
    tail = m - n_full * RPP
    chunk_lens = [RPP] * n_full + ([tail] if tail else [])
    n_chunks = len(chunk_lens)

    def body(
        x_ref,
        out_ref,
        ring_buf,
        inj_stage,
        rem_stage,
        rot_buf,
        ld_buf,
        ld_sems,
        inj_send_sems,
        inj_recv_sems,
        rem_send_sem,
        rem_recv_sem,
        cw_send_sems,
        cw_recv_sems,
        ccw_send_sems,
        ccw_recv_sems,
        ring_drain_sems,
        own_drain_sems,
        dummy_sem,
    ):
        my_x = lax.axis_index("x")
        my_y = lax.axis_index("y")
        my_z = lax.axis_index("z")
        other = 1 - my_x
        partner = (other, my_y, my_z)
        my_r = _ring_pos(my_y, my_z)
        cw_y, cw_z = _ring_coords((my_r + 1) % NP)
        ccw_y, ccw_z = _ring_coords((my_r - 1) % NP)
        cw_dev = (my_x, cw_y, cw_z)
        ccw_dev = (my_x, ccw_y, ccw_z)

        def sub_slice(piece, j):
            return pl.ds((piece * NSUB + j) * RPP, RPP)

        inj_lds = []
        for j in range(NSUB):
            ld = pltpu.make_async_copy(
                x_ref.at[pl.ds(my_r * RP + j * RPP, RPP), :],
                ld_buf.at[j],
                ld_sems.at[j],
            )
            ld.start()
            inj_lds.append(ld)

        barrier_sem = pltpu.get_barrier_semaphore()
        for dev in (partner, cw_dev, ccw_dev):
            pl.semaphore_signal(
                barrier_sem, inc=1,
                device_id=dev, device_id_type=pl.DeviceIdType.MESH,
            )
        pl.semaphore_wait(barrier_sem, 3)

        injs = []
        for j in range(NSUB):
            inj_lds[j].wait()
            inj_stage[pl.ds(j * RPP, RPP), :] = ld_buf[j].astype(jnp.bfloat16)
            r = pltpu.make_async_remote_copy(
                src_ref=inj_stage.at[pl.ds(j * RPP, RPP), :],
                dst_ref=ring_buf.at[sub_slice(my_r, j), :],
                send_sem=inj_send_sems.at[j],
                recv_sem=inj_recv_sems.at[j],
                device_id=partner,
                device_id_type=pl.DeviceIdType.MESH,
            )
            r.start()
            injs.append(r)

        order = list(range(2 * NP, n_chunks)) + list(range(2 * NP))
        chunk_lds = {}
        own_drains = []
        rem_rdma = None

        def start_ld(i):
            k = order[i]
            ln = chunk_lens[k]
            ld = pltpu.make_async_copy(
                x_ref.at[pl.ds(k * RPP, ln), :],
                ld_buf.at[i % 2, pl.ds(0, ln), :],
                ld_sems.at[i % 2],
            )
            ld.start()
            chunk_lds[i] = ld

        start_ld(0)
        start_ld(1)
        for i in range(n_chunks):
            k = order[i]
            ln = chunk_lens[k]
            chunk_lds[i].wait()
            if i >= N_ROT:
                own_drains[i - N_ROT].wait()
            if k < 2 * NP:
                rot_buf[i % N_ROT, :, :] = ld_buf[i % 2].astype(jnp.bfloat16)
                src = rot_buf.at[i % N_ROT]
            else:
                off = (k - 2 * NP) * RPP
                rem_stage[pl.ds(off, ln), :] = (
                    ld_buf[i % 2, pl.ds(0, ln), :].astype(jnp.bfloat16)
                )
                src = rem_stage.at[pl.ds(off, ln), :]
            if i + 2 < n_chunks:
                start_ld(i + 2)
            d = pltpu.make_async_copy(
                src,
                out_ref.at[pl.ds(my_x * m + k * RPP, ln), :],
                own_drain_sems.at[i % N_ROT],
            )
            d.start()
            own_drains.append(d)
            if i == n_chunks - 2 * NP - 1:
                rem_rdma = pltpu.make_async_remote_copy(
                    src_ref=rem_stage,
                    dst_ref=out_ref.at[
                        pl.ds(my_x * m + RING_ROWS, rem_rows), :
                    ],
                    send_sem=rem_send_sem,
                    recv_sem=rem_recv_sem,
                    device_id=partner,
                    device_id_type=pl.DeviceIdType.MESH,
                )
                rem_rdma.start()

        def stream_send(sem_s, sem_r, dev, piece, s, j):
            r = pltpu.make_async_remote_copy(
                src_ref=ring_buf.at[sub_slice(piece, j), :],
                dst_ref=ring_buf.at[sub_slice(piece, j), :],
                send_sem=sem_s.at[s * NSUB + j],
                recv_sem=sem_r.at[s * NSUB + j],
                device_id=dev,
                device_id_type=pl.DeviceIdType.MESH,
            )
            r.start()
            return r

        def recv_wait(sem_arr, s, j, piece):
            pltpu.make_async_remote_copy(
                src_ref=ring_buf.at[sub_slice(piece, j), :],
                dst_ref=ring_buf.at[sub_slice(piece, j), :],
                send_sem=dummy_sem,
                recv_sem=sem_arr.at[s * NSUB + j],
                device_id=cw_dev,
                device_id_type=pl.DeviceIdType.MESH,
            ).wait_recv()

        ring_drains = []

        def ring_drain(piece, j):
            i = len(ring_drains)
            if i >= N_RDRAIN:
                ring_drains[i - N_RDRAIN].wait()
            d = pltpu.make_async_copy(
                ring_buf.at[sub_slice(piece, j), :],
                out_ref.at[pl.ds(other * m + (piece * NSUB + j) * RPP, RPP), :],
                ring_drain_sems.at[i % N_RDRAIN],
            )
            d.start()
            ring_drains.append(d)

        sends = []
        for j in range(NSUB):
            injs[j].wait_recv()
            ring_drain(my_r, j)
            sends.append(stream_send(
                cw_send_sems, cw_recv_sems, cw_dev, my_r, 0, j))
            sends.append(stream_send(
                ccw_send_sems, ccw_recv_sems, ccw_dev, my_r, 0, j))
        for s in range(1, 3):
            for j in range(NSUB):
                q = (my_r - s) % NP
                recv_wait(cw_recv_sems, s - 1, j, q)
                sends.append(stream_send(
                    cw_send_sems, cw_recv_sems, cw_dev, q, s, j))
                ring_drain(q, j)
                qq = (my_r + s) % NP
                recv_wait(ccw_recv_sems, s - 1, j, qq)
                sends.append(stream_send(
                    ccw_send_sems, ccw_recv_sems, ccw_dev, qq, s, j))
                ring_drain(qq, j)
        q3 = (my_r - 3) % NP
        qq3 = (my_r + 3) % NP
        recv_wait(cw_recv_sems, 2, 0, q3)
        sends.append(stream_send(
            cw_send_sems, cw_recv_sems, cw_dev, q3, 3, 0))
        ring_drain(q3, 0)
        recv_wait(ccw_recv_sems, 2, 1, qq3)
        sends.append(stream_send(
            ccw_send_sems, ccw_recv_sems, ccw_dev, qq3, 3, 1))
        ring_drain(qq3, 1)
        recv_wait(cw_recv_sems, 2, 1, q3)
        ring_drain(q3, 1)
        recv_wait(ccw_recv_sems, 2, 0, qq3)
        ring_drain(qq3, 0)
        q4 = (my_r - 4) % NP
        recv_wait(cw_recv_sems, 3, 0, q4)
        ring_drain(q4, 0)
        recv_wait(ccw_recv_sems, 3, 1, q4)
        ring_drain(q4, 1)

        rem_rdma.wait_recv()

        for r in injs:
            r.wait_send()
        rem_rdma.wait_send()
        for snd in sends:
            snd.wait_send()
        for d in own_drains[-N_ROT:]:
            d.wait()
        for d in ring_drains[-N_RDRAIN:]:
            d.wait()

    out_shape = jax.ShapeDtypeStruct((2 * m, n), jnp.bfloat16)
    return pl.pallas_call(
        body,
        out_shape=out_shape,
        in_specs=[pl.BlockSpec(memory_space=pl.ANY)],
        out_specs=pl.BlockSpec(memory_space=pl.ANY),
        scratch_shapes=[
            pltpu.VMEM((RING_ROWS, n), jnp.bfloat16),
            pltpu.VMEM((RP, n), jnp.bfloat16),
            pltpu.VMEM((m - RING_ROWS, n), jnp.bfloat16),
            pltpu.VMEM((N_ROT, RPP, n), jnp.bfloat16),
            pltpu.VMEM((2, RPP, n), jnp.float32),
            pltpu.SemaphoreType.DMA((2,)),
            pltpu.SemaphoreType.DMA((NSUB,)),
            pltpu.SemaphoreType.DMA((NSUB,)),
            pltpu.SemaphoreType.DMA,
            pltpu.SemaphoreType.DMA,
            pltpu.SemaphoreType.DMA((N_CW * NSUB,)),
            pltpu.SemaphoreType.DMA((N_CW * NSUB,)),
            pltpu.SemaphoreType.DMA((N_CW * NSUB,)),
            pltpu.SemaphoreType.DMA((N_CW * NSUB,)),
            pltpu.SemaphoreType.DMA((N_RDRAIN,)),
            pltpu.SemaphoreType.DMA((N_ROT,)),
            pltpu.SemaphoreType.DMA,
        ],
        compiler_params=pltpu.CompilerParams(
            collective_id=0, vmem_limit_bytes=56 * 1024 * 1024
        ),
    )(x)


# device time: 192129 ns/iter; 1.0876x vs baseline; 1.0154x over previous
import jax
import jax.numpy as jnp
from jax import lax
from jax.experimental import pallas as pl
from jax.experimental.pallas import tpu as pltpu

NP = 8
NSUB = 2
RPP = 704
RP = NSUB * RPP
N_CW = 4
N_CCW = 3

RING_ROWS = NP * RP
N_ROT = 4
N_RDRAIN = 8


def _ring_pos(y, z):
    return jnp.where(y == 0, z, 7 - z)


def _ring_coords(p):
    y = jnp.where(p < 4, 0, 1)
    z = jnp.where(p < 4, p, 7 - p)
    return y, z


def kernel(x):
    m, n = x.shape
    rem_rows = m - RING_ROWS
    n_full = m // RPP
    tail = m - n_full * RPP
    chunk_lens = [RPP] * n_full + ([tail] if tail else [])
    n_chunks = len(chunk_lens)

    def body(
        x_ref,
        out_ref,
        ring_buf,
        inj_stage,
        rem_stage,
        rot_buf,
        ld_buf,
        ld_sems,
        inj_send_sems,
        inj_recv_sems,
        rem_send_sem,
        rem_recv_sem,
        cw_send_sems,
        cw_recv_sems,
        ccw_send_sems,
        ccw_recv_sems,
        ring_drain_sems,
        own_drain_sems,
        dummy_sem,
    ):
        my_x = lax.axis_index("x")
        my_y = lax.axis_index("y")
        my_z = lax.axis_index("z")
        other = 1 - my_x
        partner = (other, my_y, my_z)
        my_r = _ring_pos(my_y, my_z)
        cw_y, cw_z = _ring_coords((my_r + 1) % NP)
        ccw_y, ccw_z = _ring_coords((my_r - 1) % NP)
        cw_dev = (my_x, cw_y, cw_z)
        ccw_dev = (my_x, ccw_y, ccw_z)

        def sub_slice(piece, j):
            return pl.ds((piece * NSUB + j) * RPP, RPP)

        inj_lds = []
        for j in range(NSUB):
            ld = pltpu.make_async_copy(
                x_ref.at[pl.ds(my_r * RP + j * RPP, RPP), :],
                ld_buf.at[j],
                ld_sems.at[j],
            )
            ld.start()
            inj_lds.append(ld)

        barrier_sem = pltpu.get_barrier_semaphore()
        for dev in (partner, cw_dev, ccw_dev):
            pl.semaphore_signal(
                barrier_sem, inc=1,
                device_id=dev, device_id_type=pl.DeviceIdType.MESH,
            )
        pl.semaphore_wait(barrier_sem, 3)

        injs = []
        for j in range(NSUB):
            inj_lds[j].wait()
            inj_stage[pl.ds(j * RPP, RPP), :] = ld_buf[j].astype(jnp.bfloat16)
            r = pltpu.make_async_remote_copy(
                src_ref=inj_stage.at[pl.ds(j * RPP, RPP), :],
                dst_ref=ring_buf.at[sub_slice(my_r, j), :],
                send_sem=inj_send_sems.at[j],
                recv_sem=inj_recv_sems.at[j],
                device_id=partner,
                device_id_type=pl.DeviceIdType.MESH,
            )
            r.start()
            injs.append(r)

        order = list(range(2 * NP, n_chunks)) + list(range(2 * NP))
        chunk_lds = {}
        own_drains = []
        rem_rdma = None

        def start_ld(i):
            k = order[i]
            ln = chunk_lens[k]
            ld = pltpu.make_async_copy(
                x_ref.at[pl.ds(k * RPP, ln), :],
                ld_buf.at[i % 2, pl.ds(0, ln), :],
                ld_sems.at[i % 2],
            )
            ld.start()
            chunk_lds[i] = ld

        def process_chunk(i):
            k = order[i]
            ln = chunk_lens[k]
            chunk_lds[i].wait()
            if i >= N_ROT:
                own_drains[i - N_ROT].wait()
            if k < 2 * NP:
                rot_buf[i % N_ROT, :, :] = ld_buf[i % 2].astype(jnp.bfloat16)
                src = rot_buf.at[i % N_ROT]
            else:
                off = (k - 2 * NP) * RPP
                rem_stage[pl.ds(off, ln), :] = (
                    ld_buf[i % 2, pl.ds(0, ln), :].astype(jnp.bfloat16)
                )
                src = rem_stage.at[pl.ds(off, ln), :]
            if i + 2 < n_chunks:
                start_ld(i + 2)
            d = pltpu.make_async_copy(
                src,
                out_ref.at[pl.ds(my_x * m + k * RPP, ln), :],
                own_drain_sems.at[i % N_ROT],
            )
            d.start()
            own_drains.append(d)

        n_rem_chunks = n_chunks - 2 * NP
        next_chunk = [n_rem_chunks]

        def process_next(count):
            for _ in range(count):
                if next_chunk[0] < n_chunks:
                    process_chunk(next_chunk[0])
                    next_chunk[0] += 1

        start_ld(0)
        start_ld(1)
        for i in range(n_rem_chunks):
            process_chunk(i)
        rem_rdma = pltpu.make_async_remote_copy(
            src_ref=rem_stage,
            dst_ref=out_ref.at[pl.ds(my_x * m + RING_ROWS, rem_rows), :],
            send_sem=rem_send_sem,
            recv_sem=rem_recv_sem,
            device_id=partner,
            device_id_type=pl.DeviceIdType.MESH,
        )
        rem_rdma.start()

        def stream_send(sem_s, sem_r, dev, piece, s, j):
            r = pltpu.make_async_remote_copy(
                src_ref=ring_buf.at[sub_slice(piece, j), :],
                dst_ref=ring_buf.at[sub_slice(piece, j), :],
                send_sem=sem_s.at[s * NSUB + j],
                recv_sem=sem_r.at[s * NSUB + j],
                device_id=dev,
                device_id_type=pl.DeviceIdType.MESH,
            )
            r.start()
            return r

        def recv_wait(sem_arr, s, j, piece):
            pltpu.make_async_remote_copy(
                src_ref=ring_buf.at[sub_slice(piece, j), :],
                dst_ref=ring_buf.at[sub_slice(piece, j), :],
                send_sem=dummy_sem,
                recv_sem=sem_arr.at[s * NSUB + j],
                device_id=cw_dev,
                device_id_type=pl.DeviceIdType.MESH,
            ).wait_recv()

        ring_drains = []

        def ring_drain(piece, j):
            i = len(ring_drains)
            if i >= N_RDRAIN:
                ring_drains[i - N_RDRAIN].wait()
            d = pltpu.make_async_copy(
                ring_buf.at[sub_slice(piece, j), :],
                out_ref.at[pl.ds(other * m + (piece * NSUB + j) * RPP, RPP), :],
                ring_drain_sems.at[i % N_RDRAIN],
            )
            d.start()
            ring_drains.append(d)

        sends = []
        for j in range(NSUB):
            injs[j].wait_recv()
            ring_drain(my_r, j)
            sends.append(stream_send(
                cw_send_sems, cw_recv_sems, cw_dev, my_r, 0, j))
            sends.append(stream_send(
                ccw_send_sems, ccw_recv_sems, ccw_dev, my_r, 0, j))
        process_next(2)
        for s in range(1, 3):
            for j in range(NSUB):
                q = (my_r - s) % NP
                recv_wait(cw_recv_sems, s - 1, j, q)
                sends.append(stream_send(
                    cw_send_sems, cw_recv_sems, cw_dev, q, s, j))
                ring_drain(q, j)
                process_next(1)
                qq = (my_r + s) % NP
                recv_wait(ccw_recv_sems, s - 1, j, qq)
                sends.append(stream_send(
                    ccw_send_sems, ccw_recv_sems, ccw_dev, qq, s, j))
                ring_drain(qq, j)
                process_next(1)
        q3 = (my_r - 3) % NP
        qq3 = (my_r + 3) % NP
        recv_wait(cw_recv_sems, 2, 0, q3)
        sends.append(stream_send(
            cw_send_sems, cw_recv_sems, cw_dev, q3, 3, 0))
        ring_drain(q3, 0)
        process_next(1)
        recv_wait(ccw_recv_sems, 2, 1, qq3)
        sends.append(stream_send(
            ccw_send_sems, ccw_recv_sems, ccw_dev, qq3, 3, 1))
        ring_drain(qq3, 1)
        process_next(1)
        recv_wait(cw_recv_sems, 2, 1, q3)
        ring_drain(q3, 1)
        process_next(1)
        recv_wait(ccw_recv_sems, 2, 0, qq3)
        ring_drain(qq3, 0)
        process_next(1)
        q4 = (my_r - 4) % NP
        recv_wait(cw_recv_sems, 3, 0, q4)
        ring_drain(q4, 0)
        recv_wait(ccw_recv_sems, 3, 1, q4)
        ring_drain(q4, 1)
        process_next(n_chunks)

        rem_rdma.wait_recv()

        for r in injs:
            r.wait_send()
        rem_rdma.wait_send()
        for snd in sends:
            snd.wait_send()
        for d in own_drains[-N_ROT:]:
            d.wait()
        for d in ring_drains[-N_RDRAIN:]:
            d.wait()

    out_shape = jax.ShapeDtypeStruct((2 * m, n), jnp.bfloat16)
    return pl.pallas_call(
        body,
        out_shape=out_shape,
        in_specs=[pl.BlockSpec(memory_space=pl.ANY)],
        out_specs=pl.BlockSpec(memory_space=pl.ANY),
        scratch_shapes=[
            pltpu.VMEM((RING_ROWS, n), jnp.bfloat16),
            pltpu.VMEM((RP, n), jnp.bfloat16),
            pltpu.VMEM((m - RING_ROWS, n), jnp.bfloat16),
            pltpu.VMEM((N_ROT, RPP, n), jnp.bfloat16),
            pltpu.VMEM((2, RPP, n), jnp.float32),
            pltpu.SemaphoreType.DMA((2,)),
            pltpu.SemaphoreType.DMA((NSUB,)),
            pltpu.SemaphoreType.DMA((NSUB,)),
            pltpu.SemaphoreType.DMA,
            pltpu.SemaphoreType.DMA,
            pltpu.SemaphoreType.DMA((N_CW * NSUB,)),
            pltpu.SemaphoreType.DMA((N_CW * NSUB,)),
            pltpu.SemaphoreType.DMA((N_CW * NSUB,)),
            pltpu.SemaphoreType.DMA((N_CW * NSUB,)),
            pltpu.SemaphoreType.DMA((N_RDRAIN,)),
            pltpu.SemaphoreType.DMA((N_ROT,)),
            pltpu.SemaphoreType.DMA,
        ],
        compiler_params=pltpu.CompilerParams(
            collective_id=0, vmem_limit_bytes=56 * 1024 * 1024
        ),
    )(x)
